# baseline (speedup 1.0000x reference)
import functools

import jax
import jax.numpy as jnp
from jax import lax
from jax.experimental import pallas as pl
from jax.experimental.pallas import tpu as pltpu
from jax.experimental.pallas import tpu_sc as plsc

D = 64
SCALE = 8.0
LANES = 16
NC, NS = 2, 16
NW = NC * NS
B_TOTAL = 4096 * 200
B_PER_W = B_TOTAL // NW
CHUNK = 512
N_CHUNKS = B_PER_W // CHUNK
IDX_PER_GATHER = 128
GPC = CHUNK // IDX_PER_GATHER


def _emb_body(x_hbm, table_hbm, out_hbm, idx_v, rows_v, sem):
    wid = lax.axis_index("s") * NC + lax.axis_index("c")
    base = wid * B_PER_W

    def chunk_body(c, carry):
        off = base + c * CHUNK
        pltpu.sync_copy(x_hbm.at[pl.ds(off, CHUNK)], idx_v)
        copies = [
            pltpu.async_copy(
                table_hbm.at[idx_v.at[pl.ds(j * IDX_PER_GATHER, IDX_PER_GATHER)]],
                rows_v.at[pl.ds(j * IDX_PER_GATHER, IDX_PER_GATHER), :],
                sem,
            )
            for j in range(GPC)
        ]
        for cp in copies:
            cp.wait()

        def scale_row(r, carry2):
            for l in range(D // LANES):
                s = pl.ds(l * LANES, LANES)
                rows_v[r, s] = rows_v[r, s] * SCALE
            return carry2

        lax.fori_loop(0, CHUNK, scale_row, 0)
        pltpu.sync_copy(rows_v, out_hbm.at[pl.ds(off, CHUNK)])
        return carry

    lax.fori_loop(0, N_CHUNKS, chunk_body, 0)


@jax.jit
def kernel(x, table):
    x_flat = x.reshape(-1).astype(jnp.int32)
    run = pl.kernel(
        _emb_body,
        out_type=jax.ShapeDtypeStruct((B_TOTAL, D), jnp.float32),
        mesh=plsc.VectorSubcoreMesh(core_axis_name="c", subcore_axis_name="s"),
        scratch_types=[
            pltpu.VMEM((CHUNK,), jnp.int32),
            pltpu.VMEM((CHUNK, D), jnp.float32),
            pltpu.SemaphoreType.DMA,
        ],
        compiler_params=pltpu.CompilerParams(use_tc_tiling_on_sc=False),
    )
    out = run(x_flat, table)
    return out.reshape(x.shape[0], x.shape[1], D)


# speedup vs baseline: 1.1400x; 1.1400x over previous
import jax
import jax.numpy as jnp
from jax import lax
from jax.experimental import pallas as pl
from jax.experimental.pallas import tpu as pltpu
from jax.experimental.pallas import tpu_sc as plsc

D = 64
SCALE = 8.0
LANES = 16
NC, NS = 2, 16
NW = NC * NS
B_TOTAL = 4096 * 200
B_PER_W = B_TOTAL // NW
CHUNK = 256
N_CHUNKS = B_PER_W // CHUNK
IPG = 128
GPC = CHUNK // IPG


def _emb_body(x_hbm, table_hbm, out_hbm,
              idx_all, g0, g1, s0, s1, gsem0, gsem1, osem0, osem1):
    wid = lax.axis_index("s") * NC + lax.axis_index("c")
    base = wid * B_PER_W
    pltpu.sync_copy(x_hbm.at[pl.ds(base, B_PER_W)], idx_all)

    gbufs = (g0, g1)
    sbufs = (s0, s1)
    gsems = (gsem0, gsem1)
    osems = (osem0, osem1)

    def fire_gathers(c, b):
        for j in range(GPC):
            pltpu.async_copy(
                table_hbm.at[idx_all.at[pl.ds(c * CHUNK + j * IPG, IPG)]],
                gbufs[b].at[pl.ds(j * IPG, IPG), :],
                gsems[b],
            )

    def wait_gathers(b):
        pltpu.make_async_copy(table_hbm.at[pl.ds(0, CHUNK), :], gbufs[b],
                              gsems[b]).wait()

    def scale_buf(b):
        g = gbufs[b]
        s = sbufs[b]

        @plsc.parallel_loop(0, CHUNK, step=1, unroll=4)
        def _(r):
            for l in range(D // LANES):
                sl = pl.ds(l * LANES, LANES)
                s[r, sl] = g[r, sl] * SCALE

    def fire_store(c, b):
        pltpu.async_copy(sbufs[b], out_hbm.at[pl.ds(base + c * CHUNK, CHUNK)],
                         osems[b])

    def wait_store(b):
        pltpu.make_async_copy(sbufs[b], out_hbm.at[pl.ds(0, CHUNK)],
                              osems[b]).wait()

    fire_gathers(jnp.int32(0), 0)
    fire_gathers(jnp.int32(1), 1)

    for b in (0, 1):
        wait_gathers(b)
        scale_buf(b)
        fire_gathers(jnp.int32(b + 2), b)
        fire_store(jnp.int32(b), b)

    def pair_body(p, carry):
        for b in (0, 1):
            c = 2 * p + b
            wait_gathers(b)
            scale_buf(b)
            fire_gathers(c + 2, b)
            wait_store(b)
            fire_store(c, b)
        return carry

    lax.fori_loop(1, N_CHUNKS // 2 - 1, pair_body, 0)

    for b in (0, 1):
        wait_gathers(b)
        scale_buf(b)
        wait_store(b)
        fire_store(jnp.int32(N_CHUNKS - 2 + b), b)
    for b in (0, 1):
        wait_store(b)


@jax.jit
def kernel(x, table):
    x_flat = x.reshape(-1).astype(jnp.int32)
    run = pl.kernel(
        _emb_body,
        out_type=jax.ShapeDtypeStruct((B_TOTAL, D), jnp.float32),
        mesh=plsc.VectorSubcoreMesh(core_axis_name="c", subcore_axis_name="s"),
        scratch_types=[
            pltpu.VMEM((B_PER_W,), jnp.int32),
            pltpu.VMEM((CHUNK, D), jnp.float32),
            pltpu.VMEM((CHUNK, D), jnp.float32),
            pltpu.VMEM((CHUNK, D), jnp.float32),
            pltpu.VMEM((CHUNK, D), jnp.float32),
            pltpu.SemaphoreType.DMA,
            pltpu.SemaphoreType.DMA,
            pltpu.SemaphoreType.DMA,
            pltpu.SemaphoreType.DMA,
        ],
        compiler_params=pltpu.CompilerParams(use_tc_tiling_on_sc=False),
    )
    out = run(x_flat, table)
    return out.reshape(x.shape[0], x.shape[1], D)


# speedup vs baseline: 1.1567x; 1.0146x over previous
import jax
import jax.numpy as jnp
from jax import lax
from jax.experimental import pallas as pl
from jax.experimental.pallas import tpu as pltpu
from jax.experimental.pallas import tpu_sc as plsc

D = 64
SCALE = 8.0
LANES = 16
NC, NS = 2, 16
NW = NC * NS
B_TOTAL = 4096 * 200
B_PER_W = B_TOTAL // NW
CHUNK = 128
N_CH = B_PER_W // CHUNK


def _emb_body(jdx_hbm, poff_hbm, t2_hbm, out_hbm,
              jdx_v, poff_v, g0, g1, s0, s1,
              gsem0, gsem1, osem0, osem1):
    wid = lax.axis_index("s") * NC + lax.axis_index("c")
    base = wid * B_PER_W

    gbufs = (g0, g1)
    sbufs = (s0, s1)
    gsems = (gsem0, gsem1)
    osems = (osem0, osem1)

    pltpu.sync_copy(jdx_hbm.at[pl.ds(base, B_PER_W)], jdx_v)
    pltpu.sync_copy(poff_hbm.at[pl.ds(base, B_PER_W)], poff_v)

    def fire_gather(c, b):
        pltpu.async_copy(t2_hbm.at[jdx_v.at[pl.ds(c * CHUNK, CHUNK)]],
                         gbufs[b], gsems[b])

    def wait_gather(b):
        pltpu.make_async_copy(t2_hbm.at[pl.ds(0, CHUNK), :], gbufs[b],
                              gsems[b]).wait()

    def scale_buf(c, b):
        g = gbufs[b]
        s = sbufs[b]
        po = poff_v

        @plsc.parallel_loop(0, CHUNK // LANES, 1, unroll=1)
        def _(g16):
            t0 = g16 * LANES
            pv = po[pl.ds(c * CHUNK + t0, LANES)]
            for k in range(LANES):
                off = pv[k]
                for l in range(D // LANES):
                    s[t0 + k, pl.ds(l * LANES, LANES)] = (
                        g[t0 + k, pl.ds(off + l * LANES, LANES)] * SCALE)

    def fire_store(c, b):
        pltpu.async_copy(sbufs[b],
                         out_hbm.at[pl.ds(base + c * CHUNK, CHUNK), :],
                         osems[b])

    def wait_store(b):
        pltpu.make_async_copy(sbufs[b], out_hbm.at[pl.ds(0, CHUNK), :],
                              osems[b]).wait()

    fire_gather(jnp.int32(0), 0)
    fire_gather(jnp.int32(1), 1)

    def pair_body(p, carry):
        for b in (0, 1):
            c = 2 * p + b
            wait_gather(b)
            scale_buf(c, b)

            @pl.when(c >= 2)
            def _():
                wait_store(b)

            fire_store(c, b)

            @pl.when(c + 2 < N_CH)
            def _():
                fire_gather(c + 2, b)
        return carry

    lax.fori_loop(0, N_CH // 2, pair_body, 0)

    wait_store(0)
    wait_store(1)


@jax.jit
def kernel(x, table):
    xi = x.reshape(-1).astype(jnp.int32)
    jdx = lax.shift_right_logical(xi, 1)
    poff = (xi & 1) * D
    t2 = table.reshape(500000, 2 * D)
    run = pl.kernel(
        _emb_body,
        out_type=jax.ShapeDtypeStruct((B_TOTAL, D), jnp.float32),
        mesh=plsc.VectorSubcoreMesh(core_axis_name="c", subcore_axis_name="s"),
        scratch_types=[
            pltpu.VMEM((B_PER_W,), jnp.int32),
            pltpu.VMEM((B_PER_W,), jnp.int32),
            pltpu.VMEM((CHUNK, 2 * D), jnp.float32),
            pltpu.VMEM((CHUNK, 2 * D), jnp.float32),
            pltpu.VMEM((CHUNK, D), jnp.float32),
            pltpu.VMEM((CHUNK, D), jnp.float32),
            pltpu.SemaphoreType.DMA,
            pltpu.SemaphoreType.DMA,
            pltpu.SemaphoreType.DMA,
            pltpu.SemaphoreType.DMA,
        ],
        compiler_params=pltpu.CompilerParams(use_tc_tiling_on_sc=True),
    )
    out = run(jdx, poff, t2)
    return out.reshape(x.shape[0], x.shape[1], D)
